# baseline (speedup 1.0000x reference)
import functools
import math

import jax
import jax.numpy as jnp
from jax import lax
from jax.experimental import pallas as pl
from jax.experimental.pallas import tpu as pltpu
from jax.experimental.pallas import tpu_sc as plsc

VOCAB_ = 1000000
DIM_ = 64
SCALE_ = math.sqrt(DIM_)

NC = 2
NS = 16
NW = NC * NS
LANES = 16

NB = 4096
NJ = 200
B_TOTAL = NB * NJ
C = 256


def _sc_embed():
  b_per_w = B_TOTAL // NW
  nchunks = b_per_w // C
  assert nchunks % 2 == 0 and nchunks >= 6
  blocks_per_chunk = C // 128
  mesh = plsc.VectorSubcoreMesh(core_axis_name="c", subcore_axis_name="s")

  @functools.partial(
      pl.kernel,
      out_type=jax.ShapeDtypeStruct((NJ, 8, NB // 128, 8, 128),
                                    jnp.float32),
      mesh=mesh,
      scratch_types=[
          pltpu.VMEM((b_per_w,), jnp.int32),
          pltpu.VMEM((C, DIM_), jnp.float32),
          pltpu.VMEM((C, DIM_), jnp.float32),
          pltpu.VMEM((128, 129), jnp.float32),
          pltpu.VMEM((128, 129), jnp.float32),
          pltpu.SemaphoreType.DMA,
          pltpu.SemaphoreType.DMA,
          pltpu.SemaphoreType.DMA,
          pltpu.SemaphoreType.DMA,
      ],
      compiler_params=pltpu.CompilerParams(use_tc_tiling_on_sc=False,
                                           needs_layout_passes=False),
  )
  def embed(x_hbm, table_hbm, out_hbm, idx_all, r0, r1, t0, t1,
            sg0, sg1, ts0, ts1):
    wid = lax.axis_index("s") * NC + lax.axis_index("c")
    wbase = wid * b_per_w
    gblock0 = wid * (b_per_w // 128)

    iota16 = lax.iota(jnp.int32, 16)

    def g_start(k, rows, sem):
      pltpu.async_copy(table_hbm.at[idx_all.at[pl.ds(k * C, C)]], rows, sem)

    def g_wait(rows, sem):
      pltpu.make_async_copy(
          table_hbm.at[idx_all.at[pl.ds(0, C)]], rows, sem).wait()

    def transpose(k, rows, tbuf):
      del k
      for blk in range(blocks_per_chunk):
        drow = [jnp.int32(blk * 64 + c * 16) + iota16 for c in range(4)]

        @plsc.parallel_loop(0, 128, 1, unroll=16)
        def _(r):
          bv = jnp.broadcast_to(r, (LANES,))
          for c in range(4):
            val = rows[blk * 128 + r, pl.ds(c * 16, 16)]
            plsc.store_scatter(tbuf, [drow[c], bv], val * SCALE_)

    def s_start(k, tbuf, sem):
      for blk in range(blocks_per_chunk):
        g = gblock0 + k * blocks_per_chunk + blk
        j = lax.shift_right_logical(g, 5)
        bt = g & 31
        for dt in range(8):
          pltpu.async_copy(
              tbuf.at[pl.ds(blk * 64 + dt * 8, 8), pl.ds(0, 128)],
              out_hbm.at[j, dt, bt], sem)

    def s_wait(tbuf, sem):
      for _ in range(blocks_per_chunk * 8):
        pltpu.make_async_copy(
            tbuf.at[pl.ds(0, 8), pl.ds(0, 128)],
            out_hbm.at[0, 0, 0], sem).wait()

    pltpu.sync_copy(x_hbm.at[pl.ds(wbase, b_per_w)], idx_all)

    g_start(0, r0, sg0)

    @pl.loop(0, nchunks, step=2)
    def _(k0):
      g_wait(r0, sg0)
      g_start(k0 + 1, r1, sg1)

      @pl.when(k0 >= 2)
      def _():
        s_wait(t0, ts0)
      transpose(k0, r0, t0)
      s_start(k0, t0, ts0)

      g_wait(r1, sg1)

      @pl.when(k0 + 2 < nchunks)
      def _():
        g_start(k0 + 2, r0, sg0)

      @pl.when(k0 >= 2)
      def _():
        s_wait(t1, ts1)
      transpose(k0 + 1, r1, t1)
      s_start(k0 + 1, t1, ts1)

    s_wait(t0, ts0)
    s_wait(t1, ts1)

  return embed


def kernel(x, table):
  flat = x.T.reshape(B_TOTAL)
  out5 = _sc_embed()(flat, table)
  out = out5.transpose(2, 4, 0, 1, 3).reshape(NB, NJ, DIM_)
  return out


# speedup vs baseline: 1.0265x; 1.0265x over previous
import functools
import math

import jax
import jax.numpy as jnp
from jax import lax
from jax.experimental import pallas as pl
from jax.experimental.pallas import tpu as pltpu
from jax.experimental.pallas import tpu_sc as plsc

VOCAB_ = 1000000
DIM_ = 64
SCALE_ = math.sqrt(DIM_)

NC = 2
NS = 16
NW = NC * NS
LANES = 16

NB = 4096
NJ = 200
B_TOTAL = NB * NJ
C = 256


def _sc_embed():
  b_per_w = B_TOTAL // NW
  nchunks = b_per_w // C
  assert nchunks % 2 == 0 and nchunks >= 6
  blocks_per_chunk = C // 128
  mesh = plsc.VectorSubcoreMesh(core_axis_name="c", subcore_axis_name="s")

  @functools.partial(
      pl.kernel,
      out_type=jax.ShapeDtypeStruct((NJ, 8, NB // 128, 8, 128),
                                    jnp.float32),
      mesh=mesh,
      scratch_types=[
          pltpu.VMEM((b_per_w,), jnp.int32),
          pltpu.VMEM((C, DIM_), jnp.float32),
          pltpu.VMEM((C, DIM_), jnp.float32),
          pltpu.VMEM((128, 129), jnp.float32),
          pltpu.VMEM((128, 129), jnp.float32),
          pltpu.SemaphoreType.DMA,
          pltpu.SemaphoreType.DMA,
          pltpu.SemaphoreType.DMA,
          pltpu.SemaphoreType.DMA,
      ],
      compiler_params=pltpu.CompilerParams(use_tc_tiling_on_sc=False,
                                           needs_layout_passes=False),
  )
  def embed(x_hbm, table_hbm, out_hbm, idx_all, r0, r1, t0, t1,
            sg0, sg1, ts0, ts1):
    wid = lax.axis_index("s") * NC + lax.axis_index("c")
    wbase = wid * b_per_w
    gblock0 = wid * (b_per_w // 128)

    iota16 = lax.iota(jnp.int32, 16)

    def g_start(k, rows, sem):
      pltpu.async_copy(table_hbm.at[idx_all.at[pl.ds(k * C, C)]], rows, sem)

    def g_wait(rows, sem):
      pltpu.make_async_copy(
          table_hbm.at[idx_all.at[pl.ds(0, C)]], rows, sem).wait()

    def transpose(k, rows, tbuf):
      del k
      for blk in range(blocks_per_chunk):
        drow = [jnp.int32(blk * 64 + c * 16) + iota16 for c in range(4)]

        @plsc.parallel_loop(0, 128, 1, unroll=8)
        def _(r):
          bv = jnp.broadcast_to(r, (LANES,))
          for c in range(4):
            val = rows[blk * 128 + r, pl.ds(c * 16, 16)]
            plsc.store_scatter(tbuf, [drow[c], bv], val * SCALE_)

    def s_start(k, tbuf, sem):
      for blk in range(blocks_per_chunk):
        g = gblock0 + k * blocks_per_chunk + blk
        j = lax.shift_right_logical(g, 5)
        bt = g & 31
        for dt in range(8):
          pltpu.async_copy(
              tbuf.at[pl.ds(blk * 64 + dt * 8, 8), pl.ds(0, 128)],
              out_hbm.at[j, dt, bt], sem)

    def s_wait(tbuf, sem):
      for _ in range(blocks_per_chunk * 8):
        pltpu.make_async_copy(
            tbuf.at[pl.ds(0, 8), pl.ds(0, 128)],
            out_hbm.at[0, 0, 0], sem).wait()

    pltpu.sync_copy(x_hbm.at[pl.ds(wbase, b_per_w)], idx_all)

    g_start(0, r0, sg0)

    @pl.loop(0, nchunks, step=2)
    def _(k0):
      g_wait(r0, sg0)
      g_start(k0 + 1, r1, sg1)

      @pl.when(k0 >= 2)
      def _():
        s_wait(t0, ts0)
      transpose(k0, r0, t0)
      s_start(k0, t0, ts0)

      g_wait(r1, sg1)

      @pl.when(k0 + 2 < nchunks)
      def _():
        g_start(k0 + 2, r0, sg0)

      @pl.when(k0 >= 2)
      def _():
        s_wait(t1, ts1)
      transpose(k0 + 1, r1, t1)
      s_start(k0 + 1, t1, ts1)

    s_wait(t0, ts0)
    s_wait(t1, ts1)

  return embed


def kernel(x, table):
  flat = x.T.reshape(B_TOTAL)
  out5 = _sc_embed()(flat, table)
  out = out5.transpose(2, 4, 0, 1, 3).reshape(NB, NJ, DIM_)
  return out
